# baseline (speedup 1.0000x reference)
import functools

import jax
import jax.numpy as jnp
from jax.experimental import pallas as pl

N_PTS = 16384
N_VERTS = 6890
M_PAD = 6912
N_JOINTS = 24
K = 5
BLOCK_N = 128
BIG = 1e30


def _knn_combine_kernel(xa_ref, vt_ref, whi_ref, wlo_ref, out_ref):
    xv = xa_ref[:, :]
    vt = vt_ref[:, :]
    e0 = xv[:, 0:1] - vt[0:1, :]
    e1 = xv[:, 1:2] - vt[1:2, :]
    e2 = xv[:, 2:3] - vt[2:3, :]
    d2 = e0 * e0 + e1 * e1 + e2 * e2

    b, m = d2.shape
    out_acc = jnp.zeros((b, N_JOINTS), dtype=jnp.float32)
    denom = jnp.zeros((b,), dtype=jnp.float32)
    for _ in range(K):
        mv = jnp.min(d2, axis=1)
        conf = jnp.exp(-jnp.minimum(mv, 4.0))
        eq = d2 == mv[:, None]
        eqb = eq.astype(jnp.bfloat16)
        pm = (jnp.dot(eqb, whi_ref[:, :], preferred_element_type=jnp.float32)
              + jnp.dot(eqb, wlo_ref[:, :], preferred_element_type=jnp.float32))
        out_acc = out_acc + conf[:, None] * pm
        denom = denom + conf
        d2 = jnp.where(eq, jnp.float32(BIG), d2)

    out_ref[:, :] = out_acc / denom[:, None]


@jax.jit
def kernel(x, smpl_tfs, smpl_verts, smpl_weights):
    del smpl_tfs
    verts = smpl_verts[0]
    w = smpl_weights[0]
    pad = M_PAD - N_VERTS
    verts_p = jnp.concatenate(
        [verts, jnp.full((pad, 3), 1.0e3, dtype=verts.dtype)], axis=0)
    w_p = jnp.concatenate(
        [w, jnp.zeros((pad, N_JOINTS), dtype=w.dtype)], axis=0)
    w_hi = w_p.astype(jnp.bfloat16)
    w_lo = (w_p - w_hi.astype(jnp.float32)).astype(jnp.bfloat16)
    vt3 = verts_p.T

    grid = (N_PTS // BLOCK_N,)
    out = pl.pallas_call(
        _knn_combine_kernel,
        grid=grid,
        in_specs=[
            pl.BlockSpec((BLOCK_N, 3), lambda i: (i, 0)),
            pl.BlockSpec((3, M_PAD), lambda i: (0, 0)),
            pl.BlockSpec((M_PAD, N_JOINTS), lambda i: (0, 0)),
            pl.BlockSpec((M_PAD, N_JOINTS), lambda i: (0, 0)),
        ],
        out_specs=pl.BlockSpec((BLOCK_N, N_JOINTS), lambda i: (i, 0)),
        out_shape=jax.ShapeDtypeStruct((N_PTS, N_JOINTS), jnp.float32),
    )(x, vt3, w_hi, w_lo)
    return out[None]


# speedup vs baseline: 1.4743x; 1.4743x over previous
import functools

import jax
import jax.numpy as jnp
from jax.experimental import pallas as pl

N_PTS = 16384
N_VERTS = 6890
M_PAD = 6912
N_JOINTS = 24
K = 5
BLOCK_N = 256
BIG = 1e30


def _knn_combine_kernel(xa_ref, vt_ref, whi_ref, out_ref):
    xv = xa_ref[:, :]
    vt = vt_ref[:, :]
    e0 = xv[:, 0:1] - vt[0:1, :]
    e1 = xv[:, 1:2] - vt[1:2, :]
    e2 = xv[:, 2:3] - vt[2:3, :]
    d2 = e0 * e0 + e1 * e1 + e2 * e2

    b, m = d2.shape
    out_acc = jnp.zeros((b, N_JOINTS), dtype=jnp.float32)
    denom = jnp.zeros((b,), dtype=jnp.float32)
    for _ in range(K):
        mv = jnp.min(d2, axis=1)
        conf = jnp.exp(-jnp.minimum(mv, 4.0))
        eq = d2 == mv[:, None]
        eqf = eq.astype(jnp.float32)
        pm = jnp.dot(eqf, whi_ref[:, :], preferred_element_type=jnp.float32)
        out_acc = out_acc + conf[:, None] * pm
        denom = denom + conf
        d2 = jnp.where(eq, jnp.float32(BIG), d2)

    out_ref[:, :] = out_acc / denom[:, None]


@jax.jit
def kernel(x, smpl_tfs, smpl_verts, smpl_weights):
    del smpl_tfs
    verts = smpl_verts[0]
    w = smpl_weights[0]
    pad = M_PAD - N_VERTS
    verts_p = jnp.concatenate(
        [verts, jnp.full((pad, 3), 1.0e3, dtype=verts.dtype)], axis=0)
    w_p = jnp.concatenate(
        [w, jnp.zeros((pad, N_JOINTS), dtype=w.dtype)], axis=0)
    vt3 = verts_p.T

    grid = (N_PTS // BLOCK_N,)
    out = pl.pallas_call(
        _knn_combine_kernel,
        grid=grid,
        in_specs=[
            pl.BlockSpec((BLOCK_N, 3), lambda i: (i, 0)),
            pl.BlockSpec((3, M_PAD), lambda i: (0, 0)),
            pl.BlockSpec((M_PAD, N_JOINTS), lambda i: (0, 0)),
        ],
        out_specs=pl.BlockSpec((BLOCK_N, N_JOINTS), lambda i: (i, 0)),
        out_shape=jax.ShapeDtypeStruct((N_PTS, N_JOINTS), jnp.float32),
    )(x, vt3, w_p)
    return out[None]


# speedup vs baseline: 1.7661x; 1.1979x over previous
import functools

import jax
import jax.numpy as jnp
from jax.experimental import pallas as pl

N_PTS = 16384
N_VERTS = 6890
M_PAD = 6912
N_JOINTS = 24
K = 5
BLOCK_N = 256
BIG = 1e30


def _knn_combine_kernel(xa_ref, vt_ref, whi_ref, out_ref):
    xv = xa_ref[:, :]
    vt = vt_ref[:, :]
    e0 = xv[:, 0:1] - vt[0:1, :]
    e1 = xv[:, 1:2] - vt[1:2, :]
    e2 = xv[:, 2:3] - vt[2:3, :]
    d2 = e0 * e0 + e1 * e1 + e2 * e2

    b, m = d2.shape
    s_acc = jnp.zeros((b, m), dtype=jnp.float32)
    denom = jnp.zeros((b,), dtype=jnp.float32)
    for _ in range(K):
        mv = jnp.min(d2, axis=1)
        conf = jnp.exp(-jnp.minimum(mv, 4.0))
        eq = d2 == mv[:, None]
        s_acc = jnp.where(eq, conf[:, None], s_acc)
        denom = denom + conf
        d2 = jnp.where(eq, jnp.float32(BIG), d2)

    out = jnp.dot(s_acc, whi_ref[:, :], preferred_element_type=jnp.float32)
    out_ref[:, :] = out / denom[:, None]


@jax.jit
def kernel(x, smpl_tfs, smpl_verts, smpl_weights):
    del smpl_tfs
    verts = smpl_verts[0]
    w = smpl_weights[0]
    pad = M_PAD - N_VERTS
    verts_p = jnp.concatenate(
        [verts, jnp.full((pad, 3), 1.0e3, dtype=verts.dtype)], axis=0)
    w_p = jnp.concatenate(
        [w, jnp.zeros((pad, N_JOINTS), dtype=w.dtype)], axis=0)
    vt3 = verts_p.T

    grid = (N_PTS // BLOCK_N,)
    out = pl.pallas_call(
        _knn_combine_kernel,
        grid=grid,
        in_specs=[
            pl.BlockSpec((BLOCK_N, 3), lambda i: (i, 0)),
            pl.BlockSpec((3, M_PAD), lambda i: (0, 0)),
            pl.BlockSpec((M_PAD, N_JOINTS), lambda i: (0, 0)),
        ],
        out_specs=pl.BlockSpec((BLOCK_N, N_JOINTS), lambda i: (i, 0)),
        out_shape=jax.ShapeDtypeStruct((N_PTS, N_JOINTS), jnp.float32),
    )(x, vt3, w_p)
    return out[None]
